# Initial kernel scaffold; baseline (speedup 1.0000x reference)
import jax
import jax.numpy as jnp
from jax.experimental import pallas as pl


def kernel(x, seq_len, position_ids):
    raise NotImplementedError("write your pallas kernel here")


# speedup vs baseline: 3.9141x; 3.9141x over previous
import functools

import jax
import jax.numpy as jnp
from jax import lax
from jax.experimental import pallas as pl
from jax.experimental.pallas import tpu as pltpu
from jax.experimental.pallas import tpu_sc as plsc

DIM = 128
MAX_SEQ_LEN = 8192
THETA = 10000.0


def _build_cache():
    inv_freq = 1.0 / (THETA ** (jnp.arange(0, DIM, 2, dtype=jnp.float32) / DIM))
    t = jnp.arange(MAX_SEQ_LEN, dtype=jnp.float32)
    freqs = jnp.outer(t, inv_freq)
    emb = jnp.concatenate((freqs, freqs), axis=-1)
    return jnp.cos(emb), jnp.sin(emb)


def _make_gather(batch_total):
    info = plsc.get_sparse_core_info()
    nw = info.num_cores * info.num_subcores
    b_per_w = batch_total // nw
    chunk = 128
    n_chunks = b_per_w // chunk

    mesh = plsc.VectorSubcoreMesh(core_axis_name="c", subcore_axis_name="s")

    @functools.partial(
        pl.kernel,
        mesh=mesh,
        out_type=[
            jax.ShapeDtypeStruct((batch_total, DIM), jnp.float32),
            jax.ShapeDtypeStruct((batch_total, DIM), jnp.float32),
        ],
        scratch_types=[
            pltpu.VMEM((b_per_w,), jnp.int32),
            pltpu.VMEM((chunk, DIM), jnp.float32),
            pltpu.VMEM((chunk, DIM), jnp.float32),
            pltpu.SemaphoreType.DMA,
            pltpu.SemaphoreType.DMA,
        ],
    )
    def gather_kernel(cos_hbm, sin_hbm, idx_hbm, cos_out, sin_out,
                      idx_v, cbuf, sbuf, csem, ssem):
        wid = lax.axis_index("s") * info.num_cores + lax.axis_index("c")
        base = wid * b_per_w
        pltpu.sync_copy(idx_hbm.at[pl.ds(base, b_per_w)], idx_v)
        for c in range(n_chunks):
            rows = pl.ds(c * chunk, chunk)
            cp_c = pltpu.async_copy(cos_hbm.at[idx_v.at[rows]], cbuf, csem)
            cp_s = pltpu.async_copy(sin_hbm.at[idx_v.at[rows]], sbuf, ssem)
            cp_c.wait()
            pltpu.sync_copy(cbuf, cos_out.at[pl.ds(base + c * chunk, chunk)])
            cp_s.wait()
            pltpu.sync_copy(sbuf, sin_out.at[pl.ds(base + c * chunk, chunk)])

    return gather_kernel


def kernel(x, seq_len, position_ids):
    del x, seq_len
    cos_t, sin_t = _build_cache()
    b, s = position_ids.shape
    idx = position_ids.reshape(b * s).astype(jnp.int32)
    cos, sin = _make_gather(b * s)(cos_t, sin_t, idx)
    return cos.reshape(b, s, DIM), sin.reshape(b, s, DIM)


# speedup vs baseline: 4.2256x; 1.0796x over previous
import functools

import jax
import jax.numpy as jnp
from jax import lax
from jax.experimental import pallas as pl
from jax.experimental.pallas import tpu as pltpu
from jax.experimental.pallas import tpu_sc as plsc

DIM = 128
HALF = DIM // 2
MAX_SEQ_LEN = 8192
THETA = 10000.0


def _build_combined_table():
    inv_freq = 1.0 / (THETA ** (jnp.arange(0, DIM, 2, dtype=jnp.float32) / DIM))
    t = jnp.arange(MAX_SEQ_LEN, dtype=jnp.float32)
    freqs = jnp.outer(t, inv_freq)
    return jnp.concatenate((jnp.cos(freqs), jnp.sin(freqs)), axis=-1)


def _make_gather(batch_total):
    info = plsc.get_sparse_core_info()
    nw = info.num_cores * info.num_subcores
    b_per_w = batch_total // nw
    chunk = 128
    n_chunks = b_per_w // chunk
    nbuf = 2

    mesh = plsc.VectorSubcoreMesh(core_axis_name="c", subcore_axis_name="s")

    @functools.partial(
        pl.kernel,
        mesh=mesh,
        out_type=[
            jax.ShapeDtypeStruct((batch_total, DIM), jnp.float32),
            jax.ShapeDtypeStruct((batch_total, DIM), jnp.float32),
        ],
        scratch_types=[
            pltpu.VMEM((b_per_w,), jnp.int32),
            pltpu.VMEM((nbuf, chunk, DIM), jnp.float32),
        ]
        + [pltpu.SemaphoreType.DMA] * (2 * nbuf),
        compiler_params=pltpu.CompilerParams(use_tc_tiling_on_sc=False),
    )
    def gather_kernel(tab_hbm, idx_hbm, cos_out, sin_out,
                      idx_v, buf, gsem0, gsem1, wsem0, wsem1):
        gsems = (gsem0, gsem1)
        wsems = (wsem0, wsem1)
        wid = lax.axis_index("s") * info.num_cores + lax.axis_index("c")
        base = wid * b_per_w
        pltpu.sync_copy(idx_hbm.at[pl.ds(base, b_per_w)], idx_v)

        def gather(c):
            p = c % nbuf
            return pltpu.async_copy(
                tab_hbm.at[idx_v.at[pl.ds(c * chunk, chunk)]], buf.at[p], gsems[p])

        def writes(c):
            p = c % nbuf
            rows = pl.ds(base + c * chunk, chunk)
            lo, hi = pl.ds(0, HALF), pl.ds(HALF, HALF)
            return [
                pltpu.async_copy(buf.at[p, :, lo], cos_out.at[rows, lo], wsems[p]),
                pltpu.async_copy(buf.at[p, :, lo], cos_out.at[rows, hi], wsems[p]),
                pltpu.async_copy(buf.at[p, :, hi], sin_out.at[rows, lo], wsems[p]),
                pltpu.async_copy(buf.at[p, :, hi], sin_out.at[rows, hi], wsems[p]),
            ]

        pending_g = gather(0)
        pending_w = None
        for c in range(n_chunks):
            pending_g.wait()
            if c + 1 < n_chunks:
                if pending_w is not None:
                    for w in pending_w:
                        w.wait()
                    pending_w = None
                nxt = gather(c + 1)
            cur_w = writes(c)
            if pending_w is not None:
                for w in pending_w:
                    w.wait()
            pending_w = cur_w
            if c + 1 < n_chunks:
                pending_g = nxt
        for w in pending_w:
            w.wait()

    return gather_kernel


def kernel(x, seq_len, position_ids):
    del x, seq_len
    tab = _build_combined_table()
    b, s = position_ids.shape
    idx = position_ids.reshape(b * s).astype(jnp.int32)
    cos, sin = _make_gather(b * s)(tab, idx)
    return cos.reshape(b, s, DIM), sin.reshape(b, s, DIM)
